# baseline (speedup 1.0000x reference)
import functools

import jax
import jax.numpy as jnp
from jax import lax
from jax.experimental import pallas as pl
from jax.experimental.pallas import tpu as pltpu
from jax.experimental.pallas import tpu_sc as plsc

N = 10000
E = 320000
NC = 2
NS = 16
LANES = 16
CHUNK = 128
G = 16
NGRP = 10
NCH = G * NGRP
E_PAD = NS * NCH * CHUNK
DCH = E_PAD // (NC * NS * CHUNK)
N_ACC = 10240
ZROWS = N_ACC // NS
OCH = ZROWS // CHUNK

@functools.lru_cache(maxsize=None)
def _mesh():
    return plsc.VectorSubcoreMesh(core_axis_name="c", subcore_axis_name="s",
                                  num_cores=NC, num_subcores=NS)


def _deg_body(dst_hbm, out_hbm, idx_v, ones_v, stage_v, acc_sh):
    c = lax.axis_index("c")
    s = lax.axis_index("s")

    @pl.loop(0, CHUNK)
    def _(r):
        ones_v[pl.ds(r, 1), pl.ds(0, LANES)] = jnp.ones((1, LANES), jnp.float32)
        stage_v[pl.ds(r, 1), pl.ds(0, LANES)] = jnp.zeros((1, LANES), jnp.float32)

    @pl.loop(0, ZROWS // CHUNK)
    def _(j):
        pltpu.sync_copy(stage_v, acc_sh.at[pl.ds(s * ZROWS + j * CHUNK, CHUNK)])

    plsc.subcore_barrier()
    pltpu.sync_copy(dst_hbm.at[c, s], idx_v)

    @pl.loop(0, DCH)
    def _(j):
        pltpu.sync_copy(ones_v, acc_sh.at[idx_v.at[j]], add=True)

    plsc.subcore_barrier()

    @pl.loop(0, OCH)
    def _(j):
        base = s * ZROWS + j * CHUNK
        pltpu.sync_copy(acc_sh.at[pl.ds(base, CHUNK)], stage_v)
        pltpu.sync_copy(stage_v, out_hbm.at[c, pl.ds(base, CHUNK)])


@functools.lru_cache(maxsize=None)
def _deg_call():
  return pl.kernel(
    _deg_body,
    out_type=jax.ShapeDtypeStruct((NC, N_ACC, LANES), jnp.float32),
    mesh=_mesh(),
    compiler_params=pltpu.CompilerParams(use_tc_tiling_on_sc=False),
    scratch_types=[
        pltpu.VMEM((DCH, CHUNK), jnp.int32),
        pltpu.VMEM((CHUNK, LANES), jnp.float32),
        pltpu.VMEM((CHUNK, LANES), jnp.float32),
        pltpu.VMEM_SHARED((N_ACC, LANES), jnp.float32),
    ],
  )


@functools.lru_cache(maxsize=None)
def _prop_call(fh):
    def body(table_hbm, src_hbm, dst_hbm, out_hbm,
             src0, dst0, src1, dst1, buf_a, buf_b, acc_sh,
             sem_a, sem_b, isem):
        c = lax.axis_index("c")
        s = lax.axis_index("s")

        @pl.loop(0, CHUNK)
        def _(r):
            @pl.loop(0, fh // LANES)
            def _(k):
                buf_a[pl.ds(r, 1), pl.ds(k * LANES, LANES)] = (
                    jnp.zeros((1, LANES), jnp.float32))

        @pl.loop(0, OCH)
        def _(j):
            pltpu.sync_copy(buf_a, acc_sh.at[pl.ds(s * ZROWS + j * CHUNK, CHUNK)])

        plsc.subcore_barrier()

        pltpu.sync_copy(src_hbm.at[c, s, 0], src0)
        pltpu.sync_copy(dst_hbm.at[c, s, 0], dst0)

        def run_group(gi, sv, dv, nsv, ndv):
            @pl.when(gi + 1 < NGRP)
            def _():
                pltpu.async_copy(src_hbm.at[c, s, gi + 1], nsv, isem)
                pltpu.async_copy(dst_hbm.at[c, s, gi + 1], ndv, isem)

            pltpu.async_copy(table_hbm.at[sv.at[0]], buf_a, sem_a)

            @pl.loop(0, G // 2)
            def _(q):
                j0 = 2 * q
                pltpu.make_async_copy(
                    table_hbm.at[sv.at[j0]], buf_a, sem_a).wait()
                pltpu.async_copy(table_hbm.at[sv.at[j0 + 1]], buf_b, sem_b)
                pltpu.sync_copy(buf_a, acc_sh.at[dv.at[j0]], add=True)

                @pl.when(q < G // 2 - 1)
                def _():
                    pltpu.async_copy(table_hbm.at[sv.at[j0 + 2]], buf_a, sem_a)

                pltpu.make_async_copy(
                    table_hbm.at[sv.at[j0 + 1]], buf_b, sem_b).wait()
                pltpu.sync_copy(buf_b, acc_sh.at[dv.at[j0 + 1]], add=True)

            @pl.when(gi + 1 < NGRP)
            def _():
                pltpu.make_async_copy(src_hbm.at[c, s, 0], nsv, isem).wait()
                pltpu.make_async_copy(dst_hbm.at[c, s, 0], ndv, isem).wait()

        @pl.loop(0, NGRP // 2)
        def _(p):
            run_group(2 * p, src0, dst0, src1, dst1)
            run_group(2 * p + 1, src1, dst1, src0, dst0)

        plsc.subcore_barrier()

        @pl.loop(0, OCH)
        def _(j):
            base = s * ZROWS + j * CHUNK
            pltpu.sync_copy(acc_sh.at[pl.ds(base, CHUNK)], buf_a)
            pltpu.sync_copy(buf_a, out_hbm.at[c, pl.ds(base, CHUNK)])

    return pl.kernel(
        body,
        out_type=jax.ShapeDtypeStruct((NC, N_ACC, fh), jnp.float32),
        mesh=_mesh(),
        compiler_params=pltpu.CompilerParams(use_tc_tiling_on_sc=False),
        scratch_types=[
            pltpu.VMEM((G, CHUNK), jnp.int32),
            pltpu.VMEM((G, CHUNK), jnp.int32),
            pltpu.VMEM((G, CHUNK), jnp.int32),
            pltpu.VMEM((G, CHUNK), jnp.int32),
            pltpu.VMEM((CHUNK, fh), jnp.float32),
            pltpu.VMEM((CHUNK, fh), jnp.float32),
            pltpu.VMEM_SHARED((N_ACC, fh), jnp.float32),
            pltpu.SemaphoreType.DMA,
            pltpu.SemaphoreType.DMA,
            pltpu.SemaphoreType.DMA,
        ],
    )


RB = 400
GRID = N // RB


def _dinv_block(deg_ref):
    dsum = deg_ref[0, :, 0:1] + deg_ref[1, :, 0:1] + 1.0
    return lax.rsqrt(jnp.maximum(dsum, 1.0))


def _tc_first(x, w, deg):
    f_in, f_out = w.shape
    fh = f_out // 2

    def body(x_ref, w_ref, deg_ref, out_ref):
        dinv = _dinv_block(deg_ref)
        v = jnp.dot(x_ref[...].astype(jnp.bfloat16), w_ref[...].astype(jnp.bfloat16), preferred_element_type=jnp.float32)
        vs = v * dinv
        out_ref[0] = vs[:, :fh]
        out_ref[1] = vs[:, fh:]

    return pl.pallas_call(
        body,
        grid=(GRID,),
        in_specs=[
            pl.BlockSpec((RB, f_in), lambda i: (i, 0)),
            pl.BlockSpec((f_in, f_out), lambda i: (0, 0)),
            pl.BlockSpec((NC, RB, LANES), lambda i: (0, i, 0)),
        ],
        out_specs=pl.BlockSpec((NC, RB, fh), lambda i: (0, i, 0)),
        out_shape=jax.ShapeDtypeStruct((NC, N, fh), jnp.float32),
    )(x, w, deg)


def _tc_mid(acc, table, deg, b, w, split_in, split_out):
    f_in, f_out = w.shape
    f2 = f_in // 2
    fh = f_out // 2

    def body(acc_ref, tab_ref, deg_ref, b_ref, w_ref, out_ref):
        dinv = _dinv_block(deg_ref)
        bb = b_ref[...]
        if split_in:
            u0 = jnp.maximum((acc_ref[0] + tab_ref[0]) * dinv + bb[:, :f2], 0.0)
            u1 = jnp.maximum((acc_ref[1] + tab_ref[1]) * dinv + bb[:, f2:], 0.0)
            v = (jnp.dot(u0.astype(jnp.bfloat16), w_ref[0:f2, :].astype(jnp.bfloat16), preferred_element_type=jnp.float32)
                 + jnp.dot(u1.astype(jnp.bfloat16), w_ref[f2:, :].astype(jnp.bfloat16), preferred_element_type=jnp.float32))
        else:
            u = jnp.maximum(
                (acc_ref[0] + acc_ref[1] + tab_ref[...]) * dinv + bb, 0.0)
            v = jnp.dot(u.astype(jnp.bfloat16), w_ref[...].astype(jnp.bfloat16), preferred_element_type=jnp.float32)
        vs = v * dinv
        if split_out:
            out_ref[0] = vs[:, :fh]
            out_ref[1] = vs[:, fh:]
        else:
            out_ref[...] = vs

    acc_spec = (pl.BlockSpec((NC, RB, f2), lambda i: (0, i, 0)) if split_in
                else pl.BlockSpec((NC, RB, f_in), lambda i: (0, i, 0)))
    tab_spec = (pl.BlockSpec((NC, RB, f2), lambda i: (0, i, 0)) if split_in
                else pl.BlockSpec((RB, f_in), lambda i: (i, 0)))
    if split_out:
        out_spec = pl.BlockSpec((NC, RB, fh), lambda i: (0, i, 0))
        out_shape = jax.ShapeDtypeStruct((NC, N, fh), jnp.float32)
    else:
        out_spec = pl.BlockSpec((RB, f_out), lambda i: (i, 0))
        out_shape = jax.ShapeDtypeStruct((N, f_out), jnp.float32)

    return pl.pallas_call(
        body,
        grid=(GRID,),
        in_specs=[
            acc_spec,
            tab_spec,
            pl.BlockSpec((NC, RB, LANES), lambda i: (0, i, 0)),
            pl.BlockSpec((1, f_in), lambda i: (0, 0)),
            pl.BlockSpec((f_in, f_out), lambda i: (0, 0)),
        ],
        out_specs=out_spec,
        out_shape=out_shape,
    )(acc, table, deg, b, w)


def _tc_last(acc, table, deg, b, wr, br):
    f_in = wr.shape[0]
    f2 = f_in // 2

    def body(acc_ref, tab_ref, deg_ref, b_ref, w_ref, br_ref, out_ref):
        dinv = _dinv_block(deg_ref)
        bb = b_ref[...]
        u0 = jnp.maximum((acc_ref[0] + tab_ref[0]) * dinv + bb[:, :f2], 0.0)
        u1 = jnp.maximum((acc_ref[1] + tab_ref[1]) * dinv + bb[:, f2:], 0.0)
        v = (jnp.dot(u0.astype(jnp.bfloat16), w_ref[0:f2, :].astype(jnp.bfloat16), preferred_element_type=jnp.float32)
             + jnp.dot(u1.astype(jnp.bfloat16), w_ref[f2:, :].astype(jnp.bfloat16), preferred_element_type=jnp.float32))
        out_ref[...] = v + br_ref[...]

    return pl.pallas_call(
        body,
        grid=(GRID,),
        in_specs=[
            pl.BlockSpec((NC, RB, f2), lambda i: (0, i, 0)),
            pl.BlockSpec((NC, RB, f2), lambda i: (0, i, 0)),
            pl.BlockSpec((NC, RB, LANES), lambda i: (0, i, 0)),
            pl.BlockSpec((1, f_in), lambda i: (0, 0)),
            pl.BlockSpec((f_in, 1), lambda i: (0, 0)),
            pl.BlockSpec((1, 1), lambda i: (0, 0)),
        ],
        out_specs=pl.BlockSpec((RB, 1), lambda i: (i, 0)),
        out_shape=jax.ShapeDtypeStruct((N, 1), jnp.float32),
    )(acc, table, deg, b, wr, br)


def kernel(x, edge_index, W0, b0, W1, b1, W2, b2, W3, b3, W4, b4,
           W5, b5, W6, b6, W7, b7, Wr, br):
    src = edge_index[0]
    dst = edge_index[1]
    pad = E_PAD - E
    src_p = jnp.concatenate([src, jnp.zeros((pad,), src.dtype)])
    pad_dst = N + jnp.arange(pad, dtype=dst.dtype) % (N_ACC - N)
    dst_p = jnp.concatenate([dst, pad_dst])
    src2 = jnp.stack([src_p, src_p + N]).reshape(NC, NS, NGRP, G, CHUNK)
    dst2 = jnp.stack([dst_p, dst_p]).reshape(NC, NS, NGRP, G, CHUNK)
    dst_deg = dst_p.reshape(NC, NS, DCH, CHUNK)

    deg = _deg_call()(dst_deg)

    def prop(table):
        fh = table.shape[2]
        return _prop_call(fh)(table.reshape(NC * N, fh), src2, dst2)

    Ws = [W1, W2, W3, W4, W5, W6, W7]
    bs = [b0, b1, b2, b3, b4, b5, b6]

    table = _tc_first(x, W0, deg)
    for l in range(7):
        acc = prop(table)
        table = _tc_mid(acc, table, deg, bs[l].reshape(1, -1), Ws[l],
                        True, True)

    acc = prop(table)
    return _tc_last(acc, table, deg, b7.reshape(1, -1), Wr, br.reshape(1, 1))


# speedup vs baseline: 1.0014x; 1.0014x over previous
import functools

import jax
import jax.numpy as jnp
from jax import lax
from jax.experimental import pallas as pl
from jax.experimental.pallas import tpu as pltpu
from jax.experimental.pallas import tpu_sc as plsc

N = 10000
E = 320000
NC = 2
NS = 16
LANES = 16
CHUNK = 128
G = 16
NGRP = 10
NCH = G * NGRP
E_PAD = NS * NCH * CHUNK
DCH = E_PAD // (NC * NS * CHUNK)
N_ACC = 10240
ZROWS = N_ACC // NS
OCH = ZROWS // CHUNK

@functools.lru_cache(maxsize=None)
def _mesh():
    return plsc.VectorSubcoreMesh(core_axis_name="c", subcore_axis_name="s",
                                  num_cores=NC, num_subcores=NS)


def _deg_body(dst_hbm, out_hbm, idx_v, ones_v, stage_v, acc_sh, dsem):
    c = lax.axis_index("c")
    s = lax.axis_index("s")

    @pl.loop(0, CHUNK)
    def _(r):
        ones_v[pl.ds(r, 1), pl.ds(0, LANES)] = jnp.ones((1, LANES), jnp.float32)
        stage_v[pl.ds(r, 1), pl.ds(0, LANES)] = jnp.zeros((1, LANES), jnp.float32)

    @pl.loop(0, ZROWS // CHUNK)
    def _(j):
        pltpu.sync_copy(stage_v, acc_sh.at[pl.ds(s * ZROWS + j * CHUNK, CHUNK)])

    plsc.subcore_barrier()
    pltpu.sync_copy(dst_hbm.at[c, s], idx_v)

    @pl.loop(0, DCH)
    def _(j):
        pltpu.async_copy(ones_v, acc_sh.at[idx_v.at[j]], dsem, add=True)

    @pl.loop(0, DCH)
    def _(j):
        pltpu.make_async_copy(ones_v, acc_sh.at[idx_v.at[0]], dsem).wait()

    plsc.subcore_barrier()

    @pl.loop(0, OCH)
    def _(j):
        base = s * ZROWS + j * CHUNK
        pltpu.sync_copy(acc_sh.at[pl.ds(base, CHUNK)], stage_v)
        pltpu.sync_copy(stage_v, out_hbm.at[c, pl.ds(base, CHUNK)])


@functools.lru_cache(maxsize=None)
def _deg_call():
  return pl.kernel(
    _deg_body,
    out_type=jax.ShapeDtypeStruct((NC, N_ACC, LANES), jnp.float32),
    mesh=_mesh(),
    compiler_params=pltpu.CompilerParams(use_tc_tiling_on_sc=False),
    scratch_types=[
        pltpu.VMEM((DCH, CHUNK), jnp.int32),
        pltpu.VMEM((CHUNK, LANES), jnp.float32),
        pltpu.VMEM((CHUNK, LANES), jnp.float32),
        pltpu.VMEM_SHARED((N_ACC, LANES), jnp.float32),
        pltpu.SemaphoreType.DMA,
    ],
  )


@functools.lru_cache(maxsize=None)
def _prop_call(fh):
    def body(table_hbm, src_hbm, dst_hbm, out_hbm,
             src0, dst0, src1, dst1, buf_a, buf_b, acc_sh,
             sem_a, sem_b, isem):
        c = lax.axis_index("c")
        s = lax.axis_index("s")

        @pl.loop(0, CHUNK)
        def _(r):
            @pl.loop(0, fh // LANES)
            def _(k):
                buf_a[pl.ds(r, 1), pl.ds(k * LANES, LANES)] = (
                    jnp.zeros((1, LANES), jnp.float32))

        @pl.loop(0, OCH)
        def _(j):
            pltpu.sync_copy(buf_a, acc_sh.at[pl.ds(s * ZROWS + j * CHUNK, CHUNK)])

        plsc.subcore_barrier()

        pltpu.sync_copy(src_hbm.at[c, s, 0], src0)
        pltpu.sync_copy(dst_hbm.at[c, s, 0], dst0)

        def run_group(gi, sv, dv, nsv, ndv):
            @pl.when(gi + 1 < NGRP)
            def _():
                pltpu.async_copy(src_hbm.at[c, s, gi + 1], nsv, isem)
                pltpu.async_copy(dst_hbm.at[c, s, gi + 1], ndv, isem)

            pltpu.async_copy(table_hbm.at[sv.at[0]], buf_a, sem_a)

            @pl.loop(0, G // 2)
            def _(q):
                j0 = 2 * q
                pltpu.make_async_copy(
                    table_hbm.at[sv.at[j0]], buf_a, sem_a).wait()
                pltpu.async_copy(table_hbm.at[sv.at[j0 + 1]], buf_b, sem_b)
                pltpu.sync_copy(buf_a, acc_sh.at[dv.at[j0]], add=True)

                @pl.when(q < G // 2 - 1)
                def _():
                    pltpu.async_copy(table_hbm.at[sv.at[j0 + 2]], buf_a, sem_a)

                pltpu.make_async_copy(
                    table_hbm.at[sv.at[j0 + 1]], buf_b, sem_b).wait()
                pltpu.sync_copy(buf_b, acc_sh.at[dv.at[j0 + 1]], add=True)

            @pl.when(gi + 1 < NGRP)
            def _():
                pltpu.make_async_copy(src_hbm.at[c, s, 0], nsv, isem).wait()
                pltpu.make_async_copy(dst_hbm.at[c, s, 0], ndv, isem).wait()

        @pl.loop(0, NGRP // 2)
        def _(p):
            run_group(2 * p, src0, dst0, src1, dst1)
            run_group(2 * p + 1, src1, dst1, src0, dst0)

        plsc.subcore_barrier()

        @pl.loop(0, OCH)
        def _(j):
            base = s * ZROWS + j * CHUNK
            pltpu.sync_copy(acc_sh.at[pl.ds(base, CHUNK)], buf_a)
            pltpu.sync_copy(buf_a, out_hbm.at[c, pl.ds(base, CHUNK)])

    return pl.kernel(
        body,
        out_type=jax.ShapeDtypeStruct((NC, N_ACC, fh), jnp.float32),
        mesh=_mesh(),
        compiler_params=pltpu.CompilerParams(use_tc_tiling_on_sc=False),
        scratch_types=[
            pltpu.VMEM((G, CHUNK), jnp.int32),
            pltpu.VMEM((G, CHUNK), jnp.int32),
            pltpu.VMEM((G, CHUNK), jnp.int32),
            pltpu.VMEM((G, CHUNK), jnp.int32),
            pltpu.VMEM((CHUNK, fh), jnp.float32),
            pltpu.VMEM((CHUNK, fh), jnp.float32),
            pltpu.VMEM_SHARED((N_ACC, fh), jnp.float32),
            pltpu.SemaphoreType.DMA,
            pltpu.SemaphoreType.DMA,
            pltpu.SemaphoreType.DMA,
        ],
    )


RB = 400
GRID = N // RB


def _dinv_block(deg_ref):
    dsum = deg_ref[0, :, 0:1] + deg_ref[1, :, 0:1] + 1.0
    return lax.rsqrt(jnp.maximum(dsum, 1.0))


def _tc_first(x, w, deg):
    f_in, f_out = w.shape
    fh = f_out // 2

    def body(x_ref, w_ref, deg_ref, out_ref):
        dinv = _dinv_block(deg_ref)
        v = jnp.dot(x_ref[...].astype(jnp.bfloat16), w_ref[...].astype(jnp.bfloat16), preferred_element_type=jnp.float32)
        vs = v * dinv
        out_ref[0] = vs[:, :fh]
        out_ref[1] = vs[:, fh:]

    return pl.pallas_call(
        body,
        grid=(GRID,),
        in_specs=[
            pl.BlockSpec((RB, f_in), lambda i: (i, 0)),
            pl.BlockSpec((f_in, f_out), lambda i: (0, 0)),
            pl.BlockSpec((NC, RB, LANES), lambda i: (0, i, 0)),
        ],
        out_specs=pl.BlockSpec((NC, RB, fh), lambda i: (0, i, 0)),
        out_shape=jax.ShapeDtypeStruct((NC, N, fh), jnp.float32),
    )(x, w, deg)


def _tc_mid(acc, table, deg, b, w, split_in, split_out):
    f_in, f_out = w.shape
    f2 = f_in // 2
    fh = f_out // 2

    def body(acc_ref, tab_ref, deg_ref, b_ref, w_ref, out_ref):
        dinv = _dinv_block(deg_ref)
        bb = b_ref[...]
        if split_in:
            u0 = jnp.maximum((acc_ref[0] + tab_ref[0]) * dinv + bb[:, :f2], 0.0)
            u1 = jnp.maximum((acc_ref[1] + tab_ref[1]) * dinv + bb[:, f2:], 0.0)
            v = (jnp.dot(u0.astype(jnp.bfloat16), w_ref[0:f2, :].astype(jnp.bfloat16), preferred_element_type=jnp.float32)
                 + jnp.dot(u1.astype(jnp.bfloat16), w_ref[f2:, :].astype(jnp.bfloat16), preferred_element_type=jnp.float32))
        else:
            u = jnp.maximum(
                (acc_ref[0] + acc_ref[1] + tab_ref[...]) * dinv + bb, 0.0)
            v = jnp.dot(u.astype(jnp.bfloat16), w_ref[...].astype(jnp.bfloat16), preferred_element_type=jnp.float32)
        vs = v * dinv
        if split_out:
            out_ref[0] = vs[:, :fh]
            out_ref[1] = vs[:, fh:]
        else:
            out_ref[...] = vs

    acc_spec = (pl.BlockSpec((NC, RB, f2), lambda i: (0, i, 0)) if split_in
                else pl.BlockSpec((NC, RB, f_in), lambda i: (0, i, 0)))
    tab_spec = (pl.BlockSpec((NC, RB, f2), lambda i: (0, i, 0)) if split_in
                else pl.BlockSpec((RB, f_in), lambda i: (i, 0)))
    if split_out:
        out_spec = pl.BlockSpec((NC, RB, fh), lambda i: (0, i, 0))
        out_shape = jax.ShapeDtypeStruct((NC, N, fh), jnp.float32)
    else:
        out_spec = pl.BlockSpec((RB, f_out), lambda i: (i, 0))
        out_shape = jax.ShapeDtypeStruct((N, f_out), jnp.float32)

    return pl.pallas_call(
        body,
        grid=(GRID,),
        in_specs=[
            acc_spec,
            tab_spec,
            pl.BlockSpec((NC, RB, LANES), lambda i: (0, i, 0)),
            pl.BlockSpec((1, f_in), lambda i: (0, 0)),
            pl.BlockSpec((f_in, f_out), lambda i: (0, 0)),
        ],
        out_specs=out_spec,
        out_shape=out_shape,
    )(acc, table, deg, b, w)


def _tc_last(acc, table, deg, b, wr, br):
    f_in = wr.shape[0]
    f2 = f_in // 2

    def body(acc_ref, tab_ref, deg_ref, b_ref, w_ref, br_ref, out_ref):
        dinv = _dinv_block(deg_ref)
        bb = b_ref[...]
        u0 = jnp.maximum((acc_ref[0] + tab_ref[0]) * dinv + bb[:, :f2], 0.0)
        u1 = jnp.maximum((acc_ref[1] + tab_ref[1]) * dinv + bb[:, f2:], 0.0)
        v = (jnp.dot(u0.astype(jnp.bfloat16), w_ref[0:f2, :].astype(jnp.bfloat16), preferred_element_type=jnp.float32)
             + jnp.dot(u1.astype(jnp.bfloat16), w_ref[f2:, :].astype(jnp.bfloat16), preferred_element_type=jnp.float32))
        out_ref[...] = v + br_ref[...]

    return pl.pallas_call(
        body,
        grid=(GRID,),
        in_specs=[
            pl.BlockSpec((NC, RB, f2), lambda i: (0, i, 0)),
            pl.BlockSpec((NC, RB, f2), lambda i: (0, i, 0)),
            pl.BlockSpec((NC, RB, LANES), lambda i: (0, i, 0)),
            pl.BlockSpec((1, f_in), lambda i: (0, 0)),
            pl.BlockSpec((f_in, 1), lambda i: (0, 0)),
            pl.BlockSpec((1, 1), lambda i: (0, 0)),
        ],
        out_specs=pl.BlockSpec((RB, 1), lambda i: (i, 0)),
        out_shape=jax.ShapeDtypeStruct((N, 1), jnp.float32),
    )(acc, table, deg, b, wr, br)


def kernel(x, edge_index, W0, b0, W1, b1, W2, b2, W3, b3, W4, b4,
           W5, b5, W6, b6, W7, b7, Wr, br):
    src = edge_index[0]
    dst = edge_index[1]
    pad = E_PAD - E
    src_p = jnp.concatenate([src, jnp.zeros((pad,), src.dtype)])
    pad_dst = N + jnp.arange(pad, dtype=dst.dtype) % (N_ACC - N)
    dst_p = jnp.concatenate([dst, pad_dst])
    src2 = jnp.stack([src_p, src_p + N]).reshape(NC, NS, NGRP, G, CHUNK)
    dst2 = jnp.stack([dst_p, dst_p]).reshape(NC, NS, NGRP, G, CHUNK)
    dst_deg = dst_p.reshape(NC, NS, DCH, CHUNK)

    deg = _deg_call()(dst_deg)

    def prop(table):
        fh = table.shape[2]
        return _prop_call(fh)(table.reshape(NC * N, fh), src2, dst2)

    Ws = [W1, W2, W3, W4, W5, W6, W7]
    bs = [b0, b1, b2, b3, b4, b5, b6]

    table = _tc_first(x, W0, deg)
    for l in range(7):
        acc = prop(table)
        table = _tc_mid(acc, table, deg, bs[l].reshape(1, -1), Ws[l],
                        True, True)

    acc = prop(table)
    return _tc_last(acc, table, deg, b7.reshape(1, -1), Wr, br.reshape(1, 1))


# speedup vs baseline: 1.0409x; 1.0394x over previous
import functools

import jax
import jax.numpy as jnp
from jax import lax
from jax.experimental import pallas as pl
from jax.experimental.pallas import tpu as pltpu
from jax.experimental.pallas import tpu_sc as plsc

N = 10000
E = 320000
NC = 2
NS = 16
LANES = 16
CHUNK = 128
G = 16
NGRP = 10
NCH = G * NGRP
E_PAD = NS * NCH * CHUNK
DCH = E_PAD // (NC * NS * CHUNK)
N_ACC = 10240
ZROWS = N_ACC // NS
OCH = ZROWS // CHUNK

@functools.lru_cache(maxsize=None)
def _mesh():
    return plsc.VectorSubcoreMesh(core_axis_name="c", subcore_axis_name="s",
                                  num_cores=NC, num_subcores=NS)


def _deg_body(dst_hbm, out_hbm, idx_v, ones_v, stage_v, acc_sh, dsem):
    c = lax.axis_index("c")
    s = lax.axis_index("s")

    @pl.loop(0, CHUNK)
    def _(r):
        ones_v[pl.ds(r, 1), pl.ds(0, LANES)] = jnp.ones((1, LANES), jnp.float32)
        stage_v[pl.ds(r, 1), pl.ds(0, LANES)] = jnp.zeros((1, LANES), jnp.float32)

    @pl.loop(0, ZROWS // CHUNK)
    def _(j):
        pltpu.sync_copy(stage_v, acc_sh.at[pl.ds(s * ZROWS + j * CHUNK, CHUNK)])

    plsc.subcore_barrier()
    pltpu.sync_copy(dst_hbm.at[c, s], idx_v)

    @pl.loop(0, DCH)
    def _(j):
        pltpu.async_copy(ones_v, acc_sh.at[idx_v.at[j]], dsem, add=True)

    @pl.loop(0, DCH)
    def _(j):
        pltpu.make_async_copy(ones_v, acc_sh.at[idx_v.at[0]], dsem).wait()

    plsc.subcore_barrier()

    @pl.loop(0, OCH)
    def _(j):
        base = s * ZROWS + j * CHUNK
        pltpu.sync_copy(acc_sh.at[pl.ds(base, CHUNK)], stage_v)
        pltpu.sync_copy(stage_v, out_hbm.at[c, pl.ds(base, CHUNK)])


@functools.lru_cache(maxsize=None)
def _deg_call():
  return pl.kernel(
    _deg_body,
    out_type=jax.ShapeDtypeStruct((NC, N_ACC, LANES), jnp.float32),
    mesh=_mesh(),
    compiler_params=pltpu.CompilerParams(use_tc_tiling_on_sc=False),
    scratch_types=[
        pltpu.VMEM((DCH, CHUNK), jnp.int32),
        pltpu.VMEM((CHUNK, LANES), jnp.float32),
        pltpu.VMEM((CHUNK, LANES), jnp.float32),
        pltpu.VMEM_SHARED((N_ACC, LANES), jnp.float32),
        pltpu.SemaphoreType.DMA,
    ],
  )


@functools.lru_cache(maxsize=None)
def _prop_call(fh):
    nbuf = 2 if fh >= 128 else 4

    def body(*refs):
        table_hbm, src_hbm, dst_hbm, out_hbm, src0, dst0, src1, dst1 = refs[:8]
        bufs = refs[8:8 + nbuf]
        acc_sh = refs[8 + nbuf]
        sems = refs[9 + nbuf:9 + 2 * nbuf]
        isem = refs[9 + 2 * nbuf]
        buf_a = bufs[0]
        c = lax.axis_index("c")
        s = lax.axis_index("s")

        @pl.loop(0, CHUNK)
        def _(r):
            @pl.loop(0, fh // LANES)
            def _(k):
                buf_a[pl.ds(r, 1), pl.ds(k * LANES, LANES)] = (
                    jnp.zeros((1, LANES), jnp.float32))

        @pl.loop(0, OCH)
        def _(j):
            pltpu.sync_copy(buf_a, acc_sh.at[pl.ds(s * ZROWS + j * CHUNK, CHUNK)])

        plsc.subcore_barrier()

        pltpu.sync_copy(src_hbm.at[c, s, 0], src0)
        pltpu.sync_copy(dst_hbm.at[c, s, 0], dst0)

        def run_group(gi, sv, dv, nsv, ndv):
            @pl.when(gi + 1 < NGRP)
            def _():
                pltpu.async_copy(src_hbm.at[c, s, gi + 1], nsv, isem)
                pltpu.async_copy(dst_hbm.at[c, s, gi + 1], ndv, isem)

            for k in range(nbuf - 1):
                pltpu.async_copy(table_hbm.at[sv.at[k]], bufs[k], sems[k])

            @pl.loop(0, G // nbuf)
            def _(q):
                j0 = q * nbuf
                for b in range(nbuf):
                    j = j0 + b
                    pltpu.make_async_copy(
                        table_hbm.at[sv.at[j]], bufs[b], sems[b]).wait()
                    bn = (b + nbuf - 1) % nbuf

                    @pl.when(j + nbuf - 1 < G)
                    def _():
                        pltpu.async_copy(
                            table_hbm.at[sv.at[j + nbuf - 1]], bufs[bn],
                            sems[bn])

                    pltpu.sync_copy(bufs[b], acc_sh.at[dv.at[j]], add=True)

            @pl.when(gi + 1 < NGRP)
            def _():
                pltpu.make_async_copy(src_hbm.at[c, s, 0], nsv, isem).wait()
                pltpu.make_async_copy(dst_hbm.at[c, s, 0], ndv, isem).wait()

        @pl.loop(0, NGRP // 2)
        def _(p):
            run_group(2 * p, src0, dst0, src1, dst1)
            run_group(2 * p + 1, src1, dst1, src0, dst0)

        plsc.subcore_barrier()

        @pl.loop(0, OCH)
        def _(j):
            base = s * ZROWS + j * CHUNK
            pltpu.sync_copy(acc_sh.at[pl.ds(base, CHUNK)], buf_a)
            pltpu.sync_copy(buf_a, out_hbm.at[c, pl.ds(base, CHUNK)])

    return pl.kernel(
        body,
        out_type=jax.ShapeDtypeStruct((NC, N_ACC, fh), jnp.float32),
        mesh=_mesh(),
        compiler_params=pltpu.CompilerParams(use_tc_tiling_on_sc=False),
        scratch_types=(
            [pltpu.VMEM((G, CHUNK), jnp.int32)] * 4
            + [pltpu.VMEM((CHUNK, fh), jnp.float32)] * nbuf
            + [pltpu.VMEM_SHARED((N_ACC, fh), jnp.float32)]
            + [pltpu.SemaphoreType.DMA] * (nbuf + 1)
        ),
    )


RB = 400
GRID = N // RB


def _dinv_block(deg_ref):
    dsum = deg_ref[0, :, 0:1] + deg_ref[1, :, 0:1] + 1.0
    return lax.rsqrt(jnp.maximum(dsum, 1.0))


def _tc_first(x, w, deg):
    f_in, f_out = w.shape
    fh = f_out // 2

    def body(x_ref, w_ref, deg_ref, out_ref):
        dinv = _dinv_block(deg_ref)
        v = jnp.dot(x_ref[...].astype(jnp.bfloat16), w_ref[...].astype(jnp.bfloat16), preferred_element_type=jnp.float32)
        vs = v * dinv
        out_ref[0] = vs[:, :fh]
        out_ref[1] = vs[:, fh:]

    return pl.pallas_call(
        body,
        grid=(GRID,),
        in_specs=[
            pl.BlockSpec((RB, f_in), lambda i: (i, 0)),
            pl.BlockSpec((f_in, f_out), lambda i: (0, 0)),
            pl.BlockSpec((NC, RB, LANES), lambda i: (0, i, 0)),
        ],
        out_specs=pl.BlockSpec((NC, RB, fh), lambda i: (0, i, 0)),
        out_shape=jax.ShapeDtypeStruct((NC, N, fh), jnp.float32),
    )(x, w, deg)


def _tc_mid(acc, table, deg, b, w, split_in, split_out):
    f_in, f_out = w.shape
    f2 = f_in // 2
    fh = f_out // 2

    def body(acc_ref, tab_ref, deg_ref, b_ref, w_ref, out_ref):
        dinv = _dinv_block(deg_ref)
        bb = b_ref[...]
        if split_in:
            u0 = jnp.maximum((acc_ref[0] + tab_ref[0]) * dinv + bb[:, :f2], 0.0)
            u1 = jnp.maximum((acc_ref[1] + tab_ref[1]) * dinv + bb[:, f2:], 0.0)
            v = (jnp.dot(u0.astype(jnp.bfloat16), w_ref[0:f2, :].astype(jnp.bfloat16), preferred_element_type=jnp.float32)
                 + jnp.dot(u1.astype(jnp.bfloat16), w_ref[f2:, :].astype(jnp.bfloat16), preferred_element_type=jnp.float32))
        else:
            u = jnp.maximum(
                (acc_ref[0] + acc_ref[1] + tab_ref[...]) * dinv + bb, 0.0)
            v = jnp.dot(u.astype(jnp.bfloat16), w_ref[...].astype(jnp.bfloat16), preferred_element_type=jnp.float32)
        vs = v * dinv
        if split_out:
            out_ref[0] = vs[:, :fh]
            out_ref[1] = vs[:, fh:]
        else:
            out_ref[...] = vs

    acc_spec = (pl.BlockSpec((NC, RB, f2), lambda i: (0, i, 0)) if split_in
                else pl.BlockSpec((NC, RB, f_in), lambda i: (0, i, 0)))
    tab_spec = (pl.BlockSpec((NC, RB, f2), lambda i: (0, i, 0)) if split_in
                else pl.BlockSpec((RB, f_in), lambda i: (i, 0)))
    if split_out:
        out_spec = pl.BlockSpec((NC, RB, fh), lambda i: (0, i, 0))
        out_shape = jax.ShapeDtypeStruct((NC, N, fh), jnp.float32)
    else:
        out_spec = pl.BlockSpec((RB, f_out), lambda i: (i, 0))
        out_shape = jax.ShapeDtypeStruct((N, f_out), jnp.float32)

    return pl.pallas_call(
        body,
        grid=(GRID,),
        in_specs=[
            acc_spec,
            tab_spec,
            pl.BlockSpec((NC, RB, LANES), lambda i: (0, i, 0)),
            pl.BlockSpec((1, f_in), lambda i: (0, 0)),
            pl.BlockSpec((f_in, f_out), lambda i: (0, 0)),
        ],
        out_specs=out_spec,
        out_shape=out_shape,
    )(acc, table, deg, b, w)


def _tc_last(acc, table, deg, b, wr, br):
    f_in = wr.shape[0]
    f2 = f_in // 2

    def body(acc_ref, tab_ref, deg_ref, b_ref, w_ref, br_ref, out_ref):
        dinv = _dinv_block(deg_ref)
        bb = b_ref[...]
        u0 = jnp.maximum((acc_ref[0] + tab_ref[0]) * dinv + bb[:, :f2], 0.0)
        u1 = jnp.maximum((acc_ref[1] + tab_ref[1]) * dinv + bb[:, f2:], 0.0)
        v = (jnp.dot(u0.astype(jnp.bfloat16), w_ref[0:f2, :].astype(jnp.bfloat16), preferred_element_type=jnp.float32)
             + jnp.dot(u1.astype(jnp.bfloat16), w_ref[f2:, :].astype(jnp.bfloat16), preferred_element_type=jnp.float32))
        out_ref[...] = v + br_ref[...]

    return pl.pallas_call(
        body,
        grid=(GRID,),
        in_specs=[
            pl.BlockSpec((NC, RB, f2), lambda i: (0, i, 0)),
            pl.BlockSpec((NC, RB, f2), lambda i: (0, i, 0)),
            pl.BlockSpec((NC, RB, LANES), lambda i: (0, i, 0)),
            pl.BlockSpec((1, f_in), lambda i: (0, 0)),
            pl.BlockSpec((f_in, 1), lambda i: (0, 0)),
            pl.BlockSpec((1, 1), lambda i: (0, 0)),
        ],
        out_specs=pl.BlockSpec((RB, 1), lambda i: (i, 0)),
        out_shape=jax.ShapeDtypeStruct((N, 1), jnp.float32),
    )(acc, table, deg, b, wr, br)


def kernel(x, edge_index, W0, b0, W1, b1, W2, b2, W3, b3, W4, b4,
           W5, b5, W6, b6, W7, b7, Wr, br):
    src = edge_index[0]
    dst = edge_index[1]
    pad = E_PAD - E
    src_p = jnp.concatenate([src, jnp.zeros((pad,), src.dtype)])
    pad_dst = N + jnp.arange(pad, dtype=dst.dtype) % (N_ACC - N)
    dst_p = jnp.concatenate([dst, pad_dst])
    src2 = jnp.stack([src_p, src_p + N]).reshape(NC, NS, NGRP, G, CHUNK)
    dst2 = jnp.stack([dst_p, dst_p]).reshape(NC, NS, NGRP, G, CHUNK)
    dst_deg = dst_p.reshape(NC, NS, DCH, CHUNK)

    deg = _deg_call()(dst_deg)

    def prop(table):
        fh = table.shape[2]
        return _prop_call(fh)(table.reshape(NC * N, fh), src2, dst2)

    Ws = [W1, W2, W3, W4, W5, W6, W7]
    bs = [b0, b1, b2, b3, b4, b5, b6]

    table = _tc_first(x, W0, deg)
    for l in range(7):
        acc = prop(table)
        table = _tc_mid(acc, table, deg, bs[l].reshape(1, -1), Ws[l],
                        True, True)

    acc = prop(table)
    return _tc_last(acc, table, deg, b7.reshape(1, -1), Wr, br.reshape(1, 1))


# speedup vs baseline: 1.0436x; 1.0026x over previous
import functools

import jax
import jax.numpy as jnp
from jax import lax
from jax.experimental import pallas as pl
from jax.experimental.pallas import tpu as pltpu
from jax.experimental.pallas import tpu_sc as plsc

N = 10000
E = 320000
NC = 2
NS = 16
LANES = 16
CHUNK = 128
G = 16
NGRP = 10
NCH = G * NGRP
E_PAD = NS * NCH * CHUNK
DCH = E_PAD // (NC * NS * CHUNK)
N_ACC = 10240
ZROWS = N_ACC // NS
OCH = ZROWS // CHUNK

@functools.lru_cache(maxsize=None)
def _mesh():
    return plsc.VectorSubcoreMesh(core_axis_name="c", subcore_axis_name="s",
                                  num_cores=NC, num_subcores=NS)


def _deg_body(dst_hbm, out_hbm, idx_v, ones_v, stage_v, acc_sh, dsem):
    c = lax.axis_index("c")
    s = lax.axis_index("s")

    @pl.loop(0, CHUNK)
    def _(r):
        ones_v[pl.ds(r, 1), pl.ds(0, LANES)] = jnp.ones((1, LANES), jnp.float32)
        stage_v[pl.ds(r, 1), pl.ds(0, LANES)] = jnp.zeros((1, LANES), jnp.float32)

    @pl.loop(0, ZROWS // CHUNK)
    def _(j):
        pltpu.sync_copy(stage_v, acc_sh.at[pl.ds(s * ZROWS + j * CHUNK, CHUNK)])

    plsc.subcore_barrier()
    pltpu.sync_copy(dst_hbm.at[c, s], idx_v)

    @pl.loop(0, DCH)
    def _(j):
        pltpu.async_copy(ones_v, acc_sh.at[idx_v.at[j]], dsem, add=True)

    @pl.loop(0, DCH)
    def _(j):
        pltpu.make_async_copy(ones_v, acc_sh.at[idx_v.at[0]], dsem).wait()

    plsc.subcore_barrier()

    @pl.loop(0, OCH)
    def _(j):
        base = s * ZROWS + j * CHUNK
        pltpu.sync_copy(acc_sh.at[pl.ds(base, CHUNK)], stage_v)
        pltpu.sync_copy(stage_v, out_hbm.at[c, pl.ds(base, CHUNK)])


@functools.lru_cache(maxsize=None)
def _deg_call():
  return pl.kernel(
    _deg_body,
    out_type=jax.ShapeDtypeStruct((NC, N_ACC, LANES), jnp.float32),
    mesh=_mesh(),
    compiler_params=pltpu.CompilerParams(use_tc_tiling_on_sc=False),
    scratch_types=[
        pltpu.VMEM((DCH, CHUNK), jnp.int32),
        pltpu.VMEM((CHUNK, LANES), jnp.float32),
        pltpu.VMEM((CHUNK, LANES), jnp.float32),
        pltpu.VMEM_SHARED((N_ACC, LANES), jnp.float32),
        pltpu.SemaphoreType.DMA,
    ],
  )


@functools.lru_cache(maxsize=None)
def _prop_call(fh):
    nbuf = 2 if fh >= 128 else 4

    def body(*refs):
        table_hbm, src_hbm, dst_hbm, out_hbm, src0, dst0, src1, dst1 = refs[:8]
        bufs = refs[8:8 + nbuf]
        acc_sh = refs[8 + nbuf]
        sems = refs[9 + nbuf:9 + 2 * nbuf]
        isem = refs[9 + 2 * nbuf]
        buf_a = bufs[0]
        c = lax.axis_index("c")
        s = lax.axis_index("s")

        @pl.loop(0, CHUNK)
        def _(r):
            @pl.loop(0, fh // LANES)
            def _(k):
                buf_a[pl.ds(r, 1), pl.ds(k * LANES, LANES)] = (
                    jnp.zeros((1, LANES), jnp.float32))

        @pl.loop(0, OCH)
        def _(j):
            pltpu.sync_copy(buf_a, acc_sh.at[pl.ds(s * ZROWS + j * CHUNK, CHUNK)])

        plsc.subcore_barrier()

        pltpu.sync_copy(src_hbm.at[c, s, 0], src0)
        pltpu.sync_copy(dst_hbm.at[c, s, 0], dst0)

        def run_group(gi, sv, dv, nsv, ndv):
            @pl.when(gi + 1 < NGRP)
            def _():
                pltpu.async_copy(src_hbm.at[c, s, gi + 1], nsv, isem)
                pltpu.async_copy(dst_hbm.at[c, s, gi + 1], ndv, isem)

            for k in range(nbuf - 1):
                pltpu.async_copy(table_hbm.at[sv.at[k]], bufs[k], sems[k])

            @pl.loop(0, G // nbuf)
            def _(q):
                j0 = q * nbuf
                for b in range(nbuf):
                    j = j0 + b
                    pltpu.make_async_copy(
                        table_hbm.at[sv.at[j]], bufs[b], sems[b]).wait()
                    bn = (b + nbuf - 1) % nbuf

                    @pl.when(j + nbuf - 1 < G)
                    def _():
                        pltpu.async_copy(
                            table_hbm.at[sv.at[j + nbuf - 1]], bufs[bn],
                            sems[bn])

                    pltpu.sync_copy(bufs[b], acc_sh.at[dv.at[j]], add=True)

            @pl.when(gi + 1 < NGRP)
            def _():
                pltpu.make_async_copy(src_hbm.at[c, s, 0], nsv, isem).wait()
                pltpu.make_async_copy(dst_hbm.at[c, s, 0], ndv, isem).wait()

        @pl.loop(0, NGRP // 2)
        def _(p):
            run_group(2 * p, src0, dst0, src1, dst1)
            run_group(2 * p + 1, src1, dst1, src0, dst0)

        plsc.subcore_barrier()

        @pl.loop(0, OCH)
        def _(j):
            base = s * ZROWS + j * CHUNK
            pltpu.sync_copy(acc_sh.at[pl.ds(base, CHUNK)], buf_a)
            pltpu.sync_copy(buf_a, out_hbm.at[c, pl.ds(base, CHUNK)])

    return pl.kernel(
        body,
        out_type=jax.ShapeDtypeStruct((NC, N_ACC, fh), jnp.float32),
        mesh=_mesh(),
        compiler_params=pltpu.CompilerParams(use_tc_tiling_on_sc=False),
        scratch_types=(
            [pltpu.VMEM((G, CHUNK), jnp.int32)] * 4
            + [pltpu.VMEM((CHUNK, fh), jnp.float32)] * nbuf
            + [pltpu.VMEM_SHARED((N_ACC, fh), jnp.float32)]
            + [pltpu.SemaphoreType.DMA] * (nbuf + 1)
        ),
    )


RB = 400
GRID = N // RB


def _dinv_block(deg_ref):
    dsum = deg_ref[0, :, 0:1] + deg_ref[1, :, 0:1] + 1.0
    return lax.rsqrt(jnp.maximum(dsum, 1.0))


def _tc_first(x, w, deg):
    f_in, f_out = w.shape
    fh = f_out // 2

    def body(x_ref, w_ref, deg_ref, out_ref):
        dinv = _dinv_block(deg_ref)
        v = jnp.dot(x_ref[...], w_ref[...], preferred_element_type=jnp.float32)
        vs = v * dinv
        out_ref[0] = vs[:, :fh]
        out_ref[1] = vs[:, fh:]

    return pl.pallas_call(
        body,
        grid=(GRID,),
        in_specs=[
            pl.BlockSpec((RB, f_in), lambda i: (i, 0)),
            pl.BlockSpec((f_in, f_out), lambda i: (0, 0)),
            pl.BlockSpec((NC, RB, LANES), lambda i: (0, i, 0)),
        ],
        out_specs=pl.BlockSpec((NC, RB, fh), lambda i: (0, i, 0)),
        out_shape=jax.ShapeDtypeStruct((NC, N, fh), jnp.float32),
    )(x, w, deg)


def _tc_mid(acc, table, deg, b, w, split_in, split_out):
    f_in, f_out = w.shape
    f2 = f_in // 2
    fh = f_out // 2

    def body(acc_ref, tab_ref, deg_ref, b_ref, w_ref, out_ref):
        dinv = _dinv_block(deg_ref)
        bb = b_ref[...]
        if split_in:
            u0 = jnp.maximum((acc_ref[0] + tab_ref[0]) * dinv + bb[:, :f2], 0.0)
            u1 = jnp.maximum((acc_ref[1] + tab_ref[1]) * dinv + bb[:, f2:], 0.0)
            v = (jnp.dot(u0, w_ref[0:f2, :], preferred_element_type=jnp.float32)
                 + jnp.dot(u1, w_ref[f2:, :], preferred_element_type=jnp.float32))
        else:
            u = jnp.maximum(
                (acc_ref[0] + acc_ref[1] + tab_ref[...]) * dinv + bb, 0.0)
            v = jnp.dot(u, w_ref[...], preferred_element_type=jnp.float32)
        vs = v * dinv
        if split_out:
            out_ref[0] = vs[:, :fh]
            out_ref[1] = vs[:, fh:]
        else:
            out_ref[...] = vs

    acc_spec = (pl.BlockSpec((NC, RB, f2), lambda i: (0, i, 0)) if split_in
                else pl.BlockSpec((NC, RB, f_in), lambda i: (0, i, 0)))
    tab_spec = (pl.BlockSpec((NC, RB, f2), lambda i: (0, i, 0)) if split_in
                else pl.BlockSpec((RB, f_in), lambda i: (i, 0)))
    if split_out:
        out_spec = pl.BlockSpec((NC, RB, fh), lambda i: (0, i, 0))
        out_shape = jax.ShapeDtypeStruct((NC, N, fh), jnp.float32)
    else:
        out_spec = pl.BlockSpec((RB, f_out), lambda i: (i, 0))
        out_shape = jax.ShapeDtypeStruct((N, f_out), jnp.float32)

    return pl.pallas_call(
        body,
        grid=(GRID,),
        in_specs=[
            acc_spec,
            tab_spec,
            pl.BlockSpec((NC, RB, LANES), lambda i: (0, i, 0)),
            pl.BlockSpec((1, f_in), lambda i: (0, 0)),
            pl.BlockSpec((f_in, f_out), lambda i: (0, 0)),
        ],
        out_specs=out_spec,
        out_shape=out_shape,
    )(acc, table, deg, b, w)


def _tc_last(acc, table, deg, b, wr, br):
    f_in = wr.shape[0]
    f2 = f_in // 2

    def body(acc_ref, tab_ref, deg_ref, b_ref, w_ref, br_ref, out_ref):
        dinv = _dinv_block(deg_ref)
        bb = b_ref[...]
        u0 = jnp.maximum((acc_ref[0] + tab_ref[0]) * dinv + bb[:, :f2], 0.0)
        u1 = jnp.maximum((acc_ref[1] + tab_ref[1]) * dinv + bb[:, f2:], 0.0)
        v = (jnp.dot(u0, w_ref[0:f2, :], preferred_element_type=jnp.float32)
             + jnp.dot(u1, w_ref[f2:, :], preferred_element_type=jnp.float32))
        out_ref[...] = v + br_ref[...]

    return pl.pallas_call(
        body,
        grid=(GRID,),
        in_specs=[
            pl.BlockSpec((NC, RB, f2), lambda i: (0, i, 0)),
            pl.BlockSpec((NC, RB, f2), lambda i: (0, i, 0)),
            pl.BlockSpec((NC, RB, LANES), lambda i: (0, i, 0)),
            pl.BlockSpec((1, f_in), lambda i: (0, 0)),
            pl.BlockSpec((f_in, 1), lambda i: (0, 0)),
            pl.BlockSpec((1, 1), lambda i: (0, 0)),
        ],
        out_specs=pl.BlockSpec((RB, 1), lambda i: (i, 0)),
        out_shape=jax.ShapeDtypeStruct((N, 1), jnp.float32),
    )(acc, table, deg, b, wr, br)


def kernel(x, edge_index, W0, b0, W1, b1, W2, b2, W3, b3, W4, b4,
           W5, b5, W6, b6, W7, b7, Wr, br):
    src = edge_index[0]
    dst = edge_index[1]
    pad = E_PAD - E
    src_p = jnp.concatenate([src, jnp.zeros((pad,), src.dtype)])
    pad_dst = N + jnp.arange(pad, dtype=dst.dtype) % (N_ACC - N)
    dst_p = jnp.concatenate([dst, pad_dst])
    src2 = jnp.stack([src_p, src_p + N]).reshape(NC, NS, NGRP, G, CHUNK)
    dst2 = jnp.stack([dst_p, dst_p]).reshape(NC, NS, NGRP, G, CHUNK)
    dst_deg = dst_p.reshape(NC, NS, DCH, CHUNK)

    deg = _deg_call()(dst_deg)

    def prop(table):
        fh = table.shape[2]
        return _prop_call(fh)(table.reshape(NC * N, fh), src2, dst2)

    Ws = [W1, W2, W3, W4, W5, W6, W7]
    bs = [b0, b1, b2, b3, b4, b5, b6]

    table = _tc_first(x, W0, deg)
    for l in range(7):
        acc = prop(table)
        table = _tc_mid(acc, table, deg, bs[l].reshape(1, -1), Ws[l],
                        True, True)

    acc = prop(table)
    return _tc_last(acc, table, deg, b7.reshape(1, -1), Wr, br.reshape(1, 1))
